# baseline (speedup 1.0000x reference)
import functools

import jax
import jax.numpy as jnp
from jax import lax
from jax.experimental import pallas as pl
from jax.experimental.pallas import tpu as pltpu
from jax.experimental.pallas import tpu_sc as plsc

N = 200000
W = 128
NFULL = N // W
TAIL = N - NFULL * W
H = 20.0
BW = 25088


def _weights(t):
    s = 1.0 - t
    t2 = t * t
    t3 = t2 * t
    s2 = s * s
    s3 = s2 * s
    w0 = s3 * (1.0 / 6.0)
    w1 = 0.5 * t3 - t2 + 2.0 / 3.0
    w2 = 0.5 * s3 - s2 + 2.0 / 3.0
    w3 = t3 * (1.0 / 6.0)
    d0 = s2 * (-0.5 * H)
    d1 = (1.5 * H * t - 2.0 * H) * t
    d2 = (2.0 * H - 1.5 * H * s) * s
    d3 = t2 * (0.5 * H)
    return (w0, w1, w2, w3), (d0, d1, d2, d3)


def _frac(r):
    f = r - r.astype(jnp.int32).astype(jnp.float32)
    return jnp.where(f < 0.0, f + 1.0, f)


_mesh = plsc.VectorSubcoreMesh(core_axis_name="c", subcore_axis_name="s")


@functools.partial(
    pl.kernel,
    mesh=_mesh,
    out_type=[jax.ShapeDtypeStruct((64, N), jnp.float32)],
    scratch_types=[
        pltpu.VMEM((2, 3 * W), jnp.float32),
        pltpu.VMEM((2, 64, W), jnp.float32),
        pltpu.SemaphoreType.DMA((2,)),
        pltpu.SemaphoreType.DMA((2,)),
    ],
    compiler_params=pltpu.CompilerParams(needs_layout_passes=False),
)
def _sc_shapef(xs, ys, zs, sf_out, pv, sfb, in_sem, out_sem):
    wid = lax.axis_index("s") * 2 + lax.axis_index("c")
    nb = jnp.where(wid < 26, 49, 48)

    def in_copies(k, slot):
        c0 = (wid + 32 * k) * W
        return (
            pltpu.make_async_copy(xs.at[pl.ds(c0, W)], pv.at[slot, pl.ds(0, W)], in_sem.at[slot]),
            pltpu.make_async_copy(ys.at[pl.ds(c0, W)], pv.at[slot, pl.ds(W, W)], in_sem.at[slot]),
            pltpu.make_async_copy(zs.at[pl.ds(c0, W)], pv.at[slot, pl.ds(2 * W, W)], in_sem.at[slot]),
        )

    def out_copies(k, slot):
        c0 = (wid + 32 * k) * W
        return (
            pltpu.make_async_copy(sfb.at[slot], sf_out.at[:, pl.ds(c0, W)], out_sem.at[slot]),
        )

    def compute_groups(slot, ngroups):
        def group_body(g, carry):
            p0 = g * 16
            tx = _frac(pv[slot, pl.ds(p0, 16)] * H)
            ty = _frac(pv[slot, pl.ds(W + p0, 16)] * H)
            tz = _frac(pv[slot, pl.ds(2 * W + p0, 16)] * H)
            wx, _ = _weights(tx)
            wy, _ = _weights(ty)
            wz, _ = _weights(tz)
            for a0 in range(4):
                for a1 in range(4):
                    xy = wx[a0] * wy[a1]
                    for a2 in range(4):
                        j = a0 * 16 + a1 * 4 + a2
                        sfb[slot, j, pl.ds(p0, 16)] = xy * wz[a2]
            return carry

        lax.fori_loop(0, ngroups, group_body, 0)

    for c in in_copies(0, 0):
        c.start()

    def block_body(k, carry):
        slot = lax.rem(k, 2)

        @pl.when(k + 1 < nb)
        def _prefetch():
            for c in in_copies(k + 1, 1 - slot):
                c.start()

        for c in in_copies(k, slot):
            c.wait()

        @pl.when(k >= 2)
        def _drain_out():
            for c in out_copies(k - 2, slot):
                c.wait()

        compute_groups(slot, W // 16)
        for c in out_copies(k, slot):
            c.start()
        return carry

    lax.fori_loop(0, nb, block_body, 0)

    for c in out_copies(nb - 2, lax.rem(nb - 2, 2)):
        c.wait()
    for c in out_copies(nb - 1, lax.rem(nb - 1, 2)):
        c.wait()

    @pl.when(wid == 31)
    def _tail():
        c0 = NFULL * W
        pltpu.sync_copy(xs.at[pl.ds(c0, TAIL)], pv.at[0, pl.ds(0, TAIL)])
        pltpu.sync_copy(ys.at[pl.ds(c0, TAIL)], pv.at[0, pl.ds(W, TAIL)])
        pltpu.sync_copy(zs.at[pl.ds(c0, TAIL)], pv.at[0, pl.ds(2 * W, TAIL)])
        compute_groups(0, TAIL // 16)
        def row_copy(j, carry):
            pltpu.sync_copy(sfb.at[0, j, pl.ds(0, TAIL)], sf_out.at[j, pl.ds(c0, TAIL)])
            return carry

        lax.fori_loop(0, 64, row_copy, 0)


def _tc_grad_body(p_ref, g_ref):
    t = _frac(p_ref[...] * H)
    w, d = _weights(t)

    def rows16(vs):
        return jnp.concatenate(
            [jnp.broadcast_to(v, (16, BW)) for v in vs], axis=0
        )

    def rows4x4(vs):
        blk = jnp.concatenate([jnp.broadcast_to(v, (4, BW)) for v in vs], axis=0)
        return jnp.tile(blk, (4, 1))

    def rows1x16(vs):
        blk = jnp.stack(vs, axis=0)
        return jnp.tile(blk, (16, 1))

    WX = rows16([w[a][0] for a in range(4)])
    DWX = rows16([d[a][0] for a in range(4)])
    WY = rows4x4([w[a][1] for a in range(4)])
    DWY = rows4x4([d[a][1] for a in range(4)])
    WZ = rows1x16([w[a][2] for a in range(4)])
    DWZ = rows1x16([d[a][2] for a in range(4)])
    g_ref[0] = DWX * (WY * WZ)
    g_ref[1] = WX * (DWY * WZ)
    g_ref[2] = (WX * WY) * DWZ


_tc_grad = pl.pallas_call(
    _tc_grad_body,
    grid=(pl.cdiv(N, BW),),
    in_specs=[
        pl.BlockSpec((3, BW), lambda i: (0, i)),
    ],
    out_specs=pl.BlockSpec((3, 64, BW), lambda i: (0, 0, i)),
    out_shape=jax.ShapeDtypeStruct((3, 64, N), jnp.float32),
)


def _split_body(p_ref, x_ref, y_ref, z_ref):
    x_ref[...] = p_ref[0]
    y_ref[...] = p_ref[1]
    z_ref[...] = p_ref[2]


_tc_split = pl.pallas_call(
    _split_body,
    out_shape=[jax.ShapeDtypeStruct((N,), jnp.float32)] * 3,
)


def kernel(position_stack):
    pos_t = position_stack.astype(jnp.float32).T
    xs, ys, zs = _tc_split(pos_t)
    (sf_t,) = _sc_shapef(xs, ys, zs)
    gr_t = _tc_grad(pos_t)
    return sf_t.T, gr_t.transpose(2, 1, 0)


# speedup vs baseline: 1.0040x; 1.0040x over previous
import functools

import jax
import jax.numpy as jnp
from jax import lax
from jax.experimental import pallas as pl
from jax.experimental.pallas import tpu as pltpu
from jax.experimental.pallas import tpu_sc as plsc

N = 200000
W = 128
NFULL = N // W
TAIL = N - NFULL * W
H = 20.0
BW = 16384


def _weights(t):
    s = 1.0 - t
    t2 = t * t
    t3 = t2 * t
    s2 = s * s
    s3 = s2 * s
    w0 = s3 * (1.0 / 6.0)
    w1 = 0.5 * t3 - t2 + 2.0 / 3.0
    w2 = 0.5 * s3 - s2 + 2.0 / 3.0
    w3 = t3 * (1.0 / 6.0)
    d0 = s2 * (-0.5 * H)
    d1 = (1.5 * H * t - 2.0 * H) * t
    d2 = (2.0 * H - 1.5 * H * s) * s
    d3 = t2 * (0.5 * H)
    return (w0, w1, w2, w3), (d0, d1, d2, d3)


def _frac(r):
    f = r - r.astype(jnp.int32).astype(jnp.float32)
    return jnp.where(f < 0.0, f + 1.0, f)


_mesh = plsc.VectorSubcoreMesh(core_axis_name="c", subcore_axis_name="s")


@functools.partial(
    pl.kernel,
    mesh=_mesh,
    out_type=[jax.ShapeDtypeStruct((64, N), jnp.float32)],
    scratch_types=[
        pltpu.VMEM((2, 3 * W), jnp.float32),
        pltpu.VMEM((2, 64, W), jnp.float32),
        pltpu.SemaphoreType.DMA((2,)),
        pltpu.SemaphoreType.DMA((2,)),
    ],
    compiler_params=pltpu.CompilerParams(needs_layout_passes=False),
)
def _sc_shapef(xs, ys, zs, sf_out, pv, sfb, in_sem, out_sem):
    wid = lax.axis_index("s") * 2 + lax.axis_index("c")
    nb = jnp.where(wid < 26, 49, 48)

    def in_copies(k, slot):
        c0 = (wid + 32 * k) * W
        return (
            pltpu.make_async_copy(xs.at[pl.ds(c0, W)], pv.at[slot, pl.ds(0, W)], in_sem.at[slot]),
            pltpu.make_async_copy(ys.at[pl.ds(c0, W)], pv.at[slot, pl.ds(W, W)], in_sem.at[slot]),
            pltpu.make_async_copy(zs.at[pl.ds(c0, W)], pv.at[slot, pl.ds(2 * W, W)], in_sem.at[slot]),
        )

    def out_copies(k, slot):
        c0 = (wid + 32 * k) * W
        return (
            pltpu.make_async_copy(sfb.at[slot], sf_out.at[:, pl.ds(c0, W)], out_sem.at[slot]),
        )

    def compute_groups(slot, ngroups):
        def group_body(g, carry):
            p0 = g * 16
            tx = _frac(pv[slot, pl.ds(p0, 16)] * H)
            ty = _frac(pv[slot, pl.ds(W + p0, 16)] * H)
            tz = _frac(pv[slot, pl.ds(2 * W + p0, 16)] * H)
            wx, _ = _weights(tx)
            wy, _ = _weights(ty)
            wz, _ = _weights(tz)
            for a0 in range(4):
                for a1 in range(4):
                    xy = wx[a0] * wy[a1]
                    for a2 in range(4):
                        j = a0 * 16 + a1 * 4 + a2
                        sfb[slot, j, pl.ds(p0, 16)] = xy * wz[a2]
            return carry

        lax.fori_loop(0, ngroups, group_body, 0)

    for c in in_copies(0, 0):
        c.start()

    def block_body(k, carry):
        slot = lax.rem(k, 2)

        @pl.when(k + 1 < nb)
        def _prefetch():
            for c in in_copies(k + 1, 1 - slot):
                c.start()

        for c in in_copies(k, slot):
            c.wait()

        @pl.when(k >= 2)
        def _drain_out():
            for c in out_copies(k - 2, slot):
                c.wait()

        compute_groups(slot, W // 16)
        for c in out_copies(k, slot):
            c.start()
        return carry

    lax.fori_loop(0, nb, block_body, 0)

    for c in out_copies(nb - 2, lax.rem(nb - 2, 2)):
        c.wait()
    for c in out_copies(nb - 1, lax.rem(nb - 1, 2)):
        c.wait()

    @pl.when(wid == 31)
    def _tail():
        c0 = NFULL * W
        pltpu.sync_copy(xs.at[pl.ds(c0, TAIL)], pv.at[0, pl.ds(0, TAIL)])
        pltpu.sync_copy(ys.at[pl.ds(c0, TAIL)], pv.at[0, pl.ds(W, TAIL)])
        pltpu.sync_copy(zs.at[pl.ds(c0, TAIL)], pv.at[0, pl.ds(2 * W, TAIL)])
        compute_groups(0, TAIL // 16)
        def row_copy(j, carry):
            pltpu.sync_copy(sfb.at[0, j, pl.ds(0, TAIL)], sf_out.at[j, pl.ds(c0, TAIL)])
            return carry

        lax.fori_loop(0, 64, row_copy, 0)


def _tc_grad_body(p_ref, g_ref):
    t = _frac(p_ref[...] * H)
    w, d = _weights(t)

    def rows16(vs):
        return jnp.concatenate(
            [jnp.broadcast_to(v, (16, BW)) for v in vs], axis=0
        )

    def rows4x4(vs):
        blk = jnp.concatenate([jnp.broadcast_to(v, (4, BW)) for v in vs], axis=0)
        return jnp.tile(blk, (4, 1))

    def rows1x16(vs):
        blk = jnp.stack(vs, axis=0)
        return jnp.tile(blk, (16, 1))

    WX = rows16([w[a][0] for a in range(4)])
    DWX = rows16([d[a][0] for a in range(4)])
    WY = rows4x4([w[a][1] for a in range(4)])
    DWY = rows4x4([d[a][1] for a in range(4)])
    WZ = rows1x16([w[a][2] for a in range(4)])
    DWZ = rows1x16([d[a][2] for a in range(4)])
    g_ref[0] = DWX * (WY * WZ)
    g_ref[1] = WX * (DWY * WZ)
    g_ref[2] = (WX * WY) * DWZ


_tc_grad = pl.pallas_call(
    _tc_grad_body,
    grid=(pl.cdiv(N, BW),),
    in_specs=[
        pl.BlockSpec((3, BW), lambda i: (0, i)),
    ],
    out_specs=pl.BlockSpec((3, 64, BW), lambda i: (0, 0, i)),
    out_shape=jax.ShapeDtypeStruct((3, 64, N), jnp.float32),
)


def _split_body(p_ref, x_ref, y_ref, z_ref):
    x_ref[...] = p_ref[0]
    y_ref[...] = p_ref[1]
    z_ref[...] = p_ref[2]


_tc_split = pl.pallas_call(
    _split_body,
    out_shape=[jax.ShapeDtypeStruct((N,), jnp.float32)] * 3,
)


def kernel(position_stack):
    pos_t = position_stack.astype(jnp.float32).T
    xs, ys, zs = _tc_split(pos_t)
    (sf_t,) = _sc_shapef(xs, ys, zs)
    gr_t = _tc_grad(pos_t)
    return sf_t.T, gr_t.transpose(2, 1, 0)
